# baseline (speedup 1.0000x reference)
import functools

import jax
import jax.numpy as jnp
from jax import lax
from jax.experimental import pallas as pl
from jax.experimental.pallas import tpu as pltpu
from jax.experimental.pallas import tpu_sc as plsc

ZCH0 = 16
ZCH1 = 32
EMBED = 64
BATCH = 4096
HIST = 20

NC = 2
NS = 16
LANES = 16
NW = NC * NS
ROWS = BATCH // NW
CHUNK = ROWS * HIST
VECS = CHUNK // LANES

_SC_MESH = plsc.VectorSubcoreMesh(core_axis_name="c", subcore_axis_name="s")


@functools.partial(
    pl.kernel,
    out_type=jax.ShapeDtypeStruct((NW, LANES), jnp.float32),
    mesh=_SC_MESH,
    scratch_types=[
        pltpu.VMEM((ROWS, HIST), jnp.int32),
        pltpu.VMEM((ZCH1, EMBED), jnp.float32),
        pltpu.VMEM((ZCH1,), jnp.float32),
        pltpu.VMEM((LANES,), jnp.float32),
        pltpu.SemaphoreType.DMA,
        pltpu.SemaphoreType.DMA,
    ],
    compiler_params=pltpu.CompilerParams(
        needs_layout_passes=False, use_tc_tiling_on_sc=True),
)
def _sc_partials(ids1_hbm, w1_hbm, out_hbm,
                 ids1_v, w1_v, rs1_v, acc_v, sem_i, sem_w):
    wid = lax.axis_index("s") * NC + lax.axis_index("c")
    cp1 = pltpu.async_copy(
        ids1_hbm.at[pl.ds(wid * ROWS, ROWS), :], ids1_v, sem_i)
    cpw = pltpu.async_copy(w1_hbm, w1_v, sem_w)

    lane = lax.iota(jnp.int32, LANES)
    cpw.wait()

    def rowsums(row0):
        rows = lane + row0
        parts = [jnp.zeros((LANES,), jnp.float32) for _ in range(4)]
        for dcol in range(EMBED):
            col = jnp.full((LANES,), dcol, jnp.int32)
            parts[dcol % 4] = parts[dcol % 4] + plsc.load_gather(
                w1_v, [rows, col])
        return (parts[0] + parts[1]) + (parts[2] + parts[3])

    rs1_v[pl.ds(0, LANES)] = rowsums(0)
    rs1_v[pl.ds(LANES, LANES)] = rowsums(LANES)

    UNROLL = 8
    NACC = 4

    def body(i, carry):
        r, c = carry[0], carry[1]
        accs = list(carry[2:])
        for u in range(UNROLL):
            ids = plsc.load_gather(ids1_v, [r, c])
            accs[u % NACC] = accs[u % NACC] + plsc.load_gather(
                rs1_v, [lax.bitwise_and(ids, ZCH1 - 1)])
            c = c + LANES
            wrap = c >= HIST
            c = jnp.where(wrap, c - HIST, c)
            r = jnp.where(wrap, r + 1, r)
        return (r, c) + tuple(accs)

    zero = jnp.zeros((LANES,), jnp.float32)
    init = (jnp.zeros((LANES,), jnp.int32), lane) + (zero,) * NACC
    cp1.wait()
    out = lax.fori_loop(0, VECS // UNROLL, body, init)
    accs = out[2:]
    acc_v[...] = (accs[0] + accs[1]) + (accs[2] + accs[3])
    pltpu.sync_copy(acc_v, out_hbm.at[wid])


_POOL0_BLOCKS = 8
_BLK = BATCH // _POOL0_BLOCKS


def _tc_pool0(ids_0, W0):
    CH = 64

    def body(ids_ref, w_ref, o_ref):
        rs = jnp.sum(w_ref[...], axis=1)
        iota = lax.broadcasted_iota(jnp.int32, (ZCH0,), 0)
        vals = [jnp.sum(jnp.where(iota == r, rs, 0.0)) for r in range(ZCH0)]

        def step(i, acc):
            ids = ids_ref[pl.ds(i * CH, CH), :]
            bits = [lax.bitwise_and(ids, 1 << k) != 0 for k in range(4)]
            level = vals
            for k in range(4):
                level = [jnp.where(bits[k], level[2 * j + 1], level[2 * j])
                         for j in range(len(level) // 2)]
            return acc + level[0]

        acc = lax.fori_loop(0, BATCH // CH, step,
                            jnp.zeros((CH, HIST), jnp.float32))
        o_ref[0, 0] = jnp.sum(acc)

    return pl.pallas_call(
        body,
        out_shape=jax.ShapeDtypeStruct((1, 1), jnp.float32),
        out_specs=pl.BlockSpec(memory_space=pltpu.SMEM),
    )(ids_0, W0)


def _tc_finish(partials, p0):
    def body(p_ref, s_ref, o_ref):
        o_ref[0, 0] = (jnp.sum(p_ref[...]) + s_ref[0, 0]) * (
            1.0 / (BATCH * 2 * EMBED))

    return pl.pallas_call(
        body,
        out_shape=jax.ShapeDtypeStruct((1, 1), jnp.float32),
        in_specs=[
            pl.BlockSpec(memory_space=pltpu.VMEM),
            pl.BlockSpec(memory_space=pltpu.SMEM),
        ],
        out_specs=pl.BlockSpec(memory_space=pltpu.SMEM),
    )(partials, p0)


def kernel(ids_0, ids_1, W0, W1):
    p0 = _tc_pool0(ids_0, W0)
    partials = _sc_partials(ids_1, W1)
    return _tc_finish(partials, p0)[0, 0]


# speedup vs baseline: 1.0112x; 1.0112x over previous
import functools

import jax
import jax.numpy as jnp
from jax import lax
from jax.experimental import pallas as pl
from jax.experimental.pallas import tpu as pltpu
from jax.experimental.pallas import tpu_sc as plsc

ZCH0 = 16
ZCH1 = 32
EMBED = 64
BATCH = 4096
HIST = 20

NC = 2
NS = 16
LANES = 16
NW = NC * NS
ROWS = BATCH // NW
CHUNK = ROWS * HIST
VECS = CHUNK // LANES

_SC_MESH = plsc.VectorSubcoreMesh(core_axis_name="c", subcore_axis_name="s")


@functools.partial(
    pl.kernel,
    out_type=jax.ShapeDtypeStruct((NW, LANES), jnp.float32),
    mesh=_SC_MESH,
    scratch_types=[
        pltpu.VMEM((ROWS, HIST), jnp.int32),
        pltpu.VMEM((ZCH1, EMBED), jnp.float32),
        pltpu.VMEM((ZCH1,), jnp.float32),
        pltpu.VMEM((LANES,), jnp.float32),
        pltpu.SemaphoreType.DMA,
        pltpu.SemaphoreType.DMA,
    ],
    compiler_params=pltpu.CompilerParams(
        needs_layout_passes=False, use_tc_tiling_on_sc=True),
)
def _sc_partials(w1_hbm, ids1_hbm, out_hbm,
                 ids1_v, w1_v, rs1_v, acc_v, sem_i, sem_w):
    wid = lax.axis_index("s") * NC + lax.axis_index("c")
    cp1 = pltpu.async_copy(
        ids1_hbm.at[pl.ds(wid * ROWS, ROWS), :], ids1_v, sem_i)
    cpw = pltpu.async_copy(w1_hbm, w1_v, sem_w)

    lane = lax.iota(jnp.int32, LANES)
    cpw.wait()

    def rowsums(row0):
        rows = lane + row0
        parts = [jnp.zeros((LANES,), jnp.float32) for _ in range(4)]
        for dcol in range(EMBED):
            col = jnp.full((LANES,), dcol, jnp.int32)
            parts[dcol % 4] = parts[dcol % 4] + plsc.load_gather(
                w1_v, [rows, col])
        return (parts[0] + parts[1]) + (parts[2] + parts[3])

    rs1_v[pl.ds(0, LANES)] = rowsums(0)
    rs1_v[pl.ds(LANES, LANES)] = rowsums(LANES)

    UNROLL = 8
    NACC = 4

    def body(i, carry):
        r, c = carry[0], carry[1]
        accs = list(carry[2:])
        for u in range(UNROLL):
            ids = plsc.load_gather(ids1_v, [r, c])
            accs[u % NACC] = accs[u % NACC] + plsc.load_gather(
                rs1_v, [lax.bitwise_and(ids, ZCH1 - 1)])
            c = c + LANES
            wrap = c >= HIST
            c = jnp.where(wrap, c - HIST, c)
            r = jnp.where(wrap, r + 1, r)
        return (r, c) + tuple(accs)

    zero = jnp.zeros((LANES,), jnp.float32)
    init = (jnp.zeros((LANES,), jnp.int32), lane) + (zero,) * NACC
    cp1.wait()
    out = lax.fori_loop(0, VECS // UNROLL, body, init)
    accs = out[2:]
    acc_v[...] = (accs[0] + accs[1]) + (accs[2] + accs[3])
    pltpu.sync_copy(acc_v, out_hbm.at[wid])


_POOL0_BLOCKS = 8
_BLK = BATCH // _POOL0_BLOCKS


def _tc_pool0(ids_0, W0):
    CH = 32

    def body(ids_ref, w_ref, o_ref):
        rs = jnp.sum(w_ref[...], axis=1)
        iota = lax.broadcasted_iota(jnp.int32, (ZCH0,), 0)
        vals = [jnp.sum(jnp.where(iota == r, rs, 0.0)) for r in range(ZCH0)]

        def step(i, acc):
            ids = ids_ref[pl.ds(i * CH, CH), :]
            bits = [lax.bitwise_and(ids, 1 << k) != 0 for k in range(4)]
            level = vals
            for k in range(4):
                level = [jnp.where(bits[k], level[2 * j + 1], level[2 * j])
                         for j in range(len(level) // 2)]
            return acc + level[0]

        acc = lax.fori_loop(0, BATCH // CH, step,
                            jnp.zeros((CH, HIST), jnp.float32))
        o_ref[0, 0] = jnp.sum(acc)

    return pl.pallas_call(
        body,
        out_shape=jax.ShapeDtypeStruct((1, 1), jnp.float32),
        out_specs=pl.BlockSpec(memory_space=pltpu.SMEM),
    )(ids_0, W0)


def _tc_finish(partials, p0):
    def body(p_ref, s_ref, o_ref):
        o_ref[0, 0] = (jnp.sum(p_ref[...]) + s_ref[0, 0]) * (
            1.0 / (BATCH * 2 * EMBED))

    return pl.pallas_call(
        body,
        out_shape=jax.ShapeDtypeStruct((1, 1), jnp.float32),
        in_specs=[
            pl.BlockSpec(memory_space=pltpu.VMEM),
            pl.BlockSpec(memory_space=pltpu.SMEM),
        ],
        out_specs=pl.BlockSpec(memory_space=pltpu.SMEM),
    )(partials, p0)


def kernel(ids_0, ids_1, W0, W1):
    partials = _sc_partials(W1, ids_1)
    p0 = _tc_pool0(ids_0, W0)
    return _tc_finish(partials, p0)[0, 0]
